# speedup vs baseline: 2.9648x; 2.9648x over previous
import functools

import jax
import jax.numpy as jnp
from jax import lax
from jax.experimental import pallas as pl
from jax.experimental.pallas import tpu as pltpu
from jax.experimental.pallas import tpu_sc as plsc

BATCH = 4096
HIST = 50
DIM = 128
NUM_ROWS = 100001

_info = plsc.get_sparse_core_info()
NC, NS, L = _info.num_cores, _info.num_subcores, _info.num_lanes
NW = NC * NS
B = BATCH * HIST
B_PER_W = B // NW
JROWS = B_PER_W // DIM

_mesh = plsc.VectorSubcoreMesh(core_axis_name="c", subcore_axis_name="s")


@functools.partial(
    pl.kernel,
    mesh=_mesh,
    out_type=jax.ShapeDtypeStruct((B, DIM), jnp.float32),
    scratch_types=[
        pltpu.VMEM((JROWS, DIM), jnp.int32),
        pltpu.VMEM((DIM, DIM), jnp.float32),
        pltpu.SemaphoreType.DMA,
    ],
)
def _embed_gather(x_hbm, table_hbm, out_hbm, idx_v, rows_v, sem):
    wid = lax.axis_index("s") * NC + lax.axis_index("c")
    base = wid * B_PER_W

    pltpu.sync_copy(x_hbm.at[wid], idx_v)

    def add_row(j, _):
        def add_lane(l, _):
            sl = pl.ds(l * L, L)
            idx_v[j, sl] = idx_v[j, sl] + 1
            return 0
        return lax.fori_loop(0, DIM // L, add_lane, 0)

    lax.fori_loop(0, JROWS, add_row, 0)

    def gather_row(j, _):
        pltpu.async_copy(table_hbm.at[idx_v.at[j]], rows_v, sem).wait()
        pltpu.sync_copy(rows_v, out_hbm.at[pl.ds(base + j * DIM, DIM)])
        return 0

    lax.fori_loop(0, JROWS, gather_row, 0)


def kernel(x, table):
    x_r = x.astype(jnp.int32).reshape(NW, JROWS, DIM)
    out = _embed_gather(x_r, table)
    return out.reshape(BATCH, HIST, DIM)


# speedup vs baseline: 3.3194x; 1.1196x over previous
import functools

import jax
import jax.numpy as jnp
from jax import lax
from jax.experimental import pallas as pl
from jax.experimental.pallas import tpu as pltpu
from jax.experimental.pallas import tpu_sc as plsc

BATCH = 4096
HIST = 50
DIM = 128
NUM_ROWS = 100001

_info = plsc.get_sparse_core_info()
NC, NS, L = _info.num_cores, _info.num_subcores, _info.num_lanes
NW = NC * NS
B = BATCH * HIST
B_PER_W = B // NW
JROWS = B_PER_W // DIM

_mesh = plsc.VectorSubcoreMesh(core_axis_name="c", subcore_axis_name="s")

NBUF = 5
NGROUPS = JROWS // NBUF


@functools.partial(
    pl.kernel,
    mesh=_mesh,
    out_type=jax.ShapeDtypeStruct((B, DIM), jnp.float32),
    scratch_types=(
        [pltpu.VMEM((JROWS, DIM), jnp.int32)]
        + [pltpu.VMEM((DIM, DIM), jnp.float32)] * NBUF
        + [pltpu.SemaphoreType.DMA] * (2 * NBUF)
    ),
)
def _embed_gather(x_hbm, table_hbm, out_hbm, idx_v, *bufs):
    rows = bufs[:NBUF]
    gsem = bufs[NBUF:2 * NBUF]
    wsem = bufs[2 * NBUF:]
    wid = lax.axis_index("s") * NC + lax.axis_index("c")
    base = wid * B_PER_W

    pltpu.sync_copy(x_hbm.at[wid], idx_v)

    def add_block(j):
        for l in range(DIM // L):
            sl = pl.ds(l * L, L)
            idx_v[j, sl] = idx_v[j, sl] + 1

    def issue_gather(j, b):
        pltpu.async_copy(table_hbm.at[idx_v.at[j]], rows[b], gsem[b])

    def wait_gather(b):
        pltpu.make_async_copy(table_hbm.at[pl.ds(0, DIM)], rows[b], gsem[b]).wait()

    def issue_write(j, b):
        pltpu.async_copy(rows[b], out_hbm.at[pl.ds(base + j * DIM, DIM)], wsem[b])

    def wait_write(b):
        pltpu.make_async_copy(rows[b], out_hbm.at[pl.ds(0, DIM)], wsem[b]).wait()

    for b in range(NBUF):
        add_block(b)
    for b in range(NBUF):
        issue_gather(b, b)

    def group_body(g, _):
        for b in range(NBUF):
            add_block((g + 1) * NBUF + b)
        for b in range(NBUF):
            wait_gather(b)
            issue_write(g * NBUF + b, b)
        for b in range(NBUF):
            wait_write(b)
            issue_gather((g + 1) * NBUF + b, b)
        return 0

    lax.fori_loop(0, NGROUPS - 1, group_body, 0)

    for b in range(NBUF):
        wait_gather(b)
        issue_write((NGROUPS - 1) * NBUF + b, b)
    for b in range(NBUF):
        wait_write(b)


def kernel(x, table):
    x_r = x.astype(jnp.int32).reshape(NW, JROWS, DIM)
    out = _embed_gather(x_r, table)
    return out.reshape(BATCH, HIST, DIM)


# speedup vs baseline: 5.2722x; 1.5883x over previous
import functools

import jax
import jax.numpy as jnp
from jax import lax
from jax.experimental import pallas as pl
from jax.experimental.pallas import tpu as pltpu
from jax.experimental.pallas import tpu_sc as plsc

BATCH = 4096
HIST = 50
DIM = 128
HPAD = 56

_info = plsc.get_sparse_core_info()
NC, NS, L = _info.num_cores, _info.num_subcores, _info.num_lanes
NW = NC * NS
B = BATCH * HIST
B_PER_W = B // NW
BATCH_PER_W = BATCH // NW
CHUNK_B = 2
CHUNK_ROWS = CHUNK_B * HPAD
NCHUNK = BATCH_PER_W // CHUNK_B
NBUF = 4
NGROUPS = NCHUNK // NBUF

_mesh = plsc.VectorSubcoreMesh(core_axis_name="c", subcore_axis_name="s")


@functools.partial(
    pl.kernel,
    mesh=_mesh,
    out_type=jax.ShapeDtypeStruct((BATCH, HIST, DIM), jnp.float32),
    scratch_types=(
        [
            pltpu.VMEM((B_PER_W + 4 * L,), jnp.int32),
            pltpu.VMEM((BATCH_PER_W * HPAD + L,), jnp.int32),
        ]
        + [pltpu.VMEM((CHUNK_ROWS, DIM), jnp.float32)] * NBUF
        + [pltpu.SemaphoreType.DMA] * (2 * NBUF)
    ),
    compiler_params=pltpu.CompilerParams(
        use_tc_tiling_on_sc=True, needs_layout_passes=False),
)
def _embed_gather(x_hbm, table_hbm, out_hbm, idx_flat, idx_str, *bufs):
    rows = bufs[:NBUF]
    gsem = bufs[NBUF:2 * NBUF]
    wsem = bufs[2 * NBUF:]
    wid = lax.axis_index("s") * NC + lax.axis_index("c")
    base_b = wid * BATCH_PER_W

    pltpu.sync_copy(x_hbm.at[pl.ds(wid * B_PER_W, B_PER_W)],
                    idx_flat.at[pl.ds(0, B_PER_W)])
    zeros = jnp.zeros((L,), jnp.int32)
    for t in range(4):
        idx_flat[pl.ds(B_PER_W + t * L, L)] = zeros

    lanes = lax.iota(jnp.int32, L)

    def rearrange(b, _):
        src = b * HIST
        dst = b * HPAD
        for l in range(4):
            v = plsc.load_gather(idx_flat, [lanes + (src + l * L)])
            idx_str[pl.ds(dst + l * L, L)] = v + 1
        return 0

    lax.fori_loop(0, BATCH_PER_W, rearrange, 0)

    def issue_gather(c, b):
        pltpu.async_copy(
            table_hbm.at[idx_str.at[pl.ds(c * CHUNK_ROWS, CHUNK_ROWS)]],
            rows[b], gsem[b])

    def wait_gather(b):
        pltpu.make_async_copy(
            table_hbm.at[pl.ds(0, CHUNK_ROWS)], rows[b], gsem[b]).wait()

    def issue_writes(c, b):
        b0 = base_b + c * CHUNK_B
        pltpu.async_copy(rows[b].at[pl.ds(0, HIST)], out_hbm.at[b0], wsem[b])
        pltpu.async_copy(rows[b].at[pl.ds(HPAD, HIST)], out_hbm.at[b0 + 1],
                         wsem[b])

    def wait_writes(b):
        for _ in range(CHUNK_B):
            pltpu.make_async_copy(
                rows[b].at[pl.ds(0, HIST)], out_hbm.at[0], wsem[b]).wait()

    for b in range(NBUF):
        issue_gather(b, b)

    def group_body(g, _):
        for b in range(NBUF):
            wait_gather(b)
            issue_writes(g * NBUF + b, b)
        for b in range(NBUF):
            wait_writes(b)
            issue_gather((g + 1) * NBUF + b, b)
        return 0

    lax.fori_loop(0, NGROUPS - 1, group_body, 0)

    for b in range(NBUF):
        wait_gather(b)
        issue_writes((NGROUPS - 1) * NBUF + b, b)
    for b in range(NBUF):
        wait_writes(b)


def kernel(x, table):
    x_flat = x.astype(jnp.int32).reshape(B)
    return _embed_gather(x_flat, table)


# speedup vs baseline: 9.8481x; 1.8679x over previous
import functools

import jax
import jax.numpy as jnp
from jax import lax
from jax.experimental import pallas as pl
from jax.experimental.pallas import tpu as pltpu
from jax.experimental.pallas import tpu_sc as plsc

BATCH = 4096
HIST = 50
DIM = 128

_info = plsc.get_sparse_core_info()
NC, NS, L = _info.num_cores, _info.num_subcores, _info.num_lanes
NW = NC * NS
B = BATCH * HIST
B_PER_W = B // NW
BATCH_PER_W = BATCH // NW
NBUF = 5
NGROUPS = HIST // NBUF

_mesh = plsc.VectorSubcoreMesh(core_axis_name="c", subcore_axis_name="s")


@functools.partial(
    pl.kernel,
    mesh=_mesh,
    out_type=jax.ShapeDtypeStruct((HIST, BATCH, DIM), jnp.float32),
    scratch_types=(
        [
            pltpu.VMEM((B_PER_W,), jnp.int32),
            pltpu.VMEM((B_PER_W,), jnp.int32),
        ]
        + [pltpu.VMEM((BATCH_PER_W, DIM), jnp.float32)] * NBUF
        + [pltpu.SemaphoreType.DMA] * (2 * NBUF)
    ),
    compiler_params=pltpu.CompilerParams(needs_layout_passes=False),
)
def _embed_gather(x_hbm, table_hbm, out_hbm, idx_flat, idx_str, *bufs):
    rows = bufs[:NBUF]
    gsem = bufs[NBUF:2 * NBUF]
    wsem = bufs[2 * NBUF:]
    wid = lax.axis_index("s") * NC + lax.axis_index("c")
    base_b = wid * BATCH_PER_W

    pltpu.sync_copy(x_hbm.at[pl.ds(wid * B_PER_W, B_PER_W)], idx_flat)

    lanes_h = lax.iota(jnp.int32, L) * HIST

    def rearrange(h, _):
        for l in range(BATCH_PER_W // L):
            src = lanes_h + (l * L * HIST + h)
            v = plsc.load_gather(idx_flat, [src])
            idx_str[pl.ds(h * BATCH_PER_W + l * L, L)] = v + 1
        return 0

    lax.fori_loop(0, HIST, rearrange, 0)

    def issue_gather(h, b):
        pltpu.async_copy(
            table_hbm.at[idx_str.at[pl.ds(h * BATCH_PER_W, BATCH_PER_W)]],
            rows[b], gsem[b])

    def wait_gather(b):
        pltpu.make_async_copy(
            table_hbm.at[pl.ds(0, BATCH_PER_W)], rows[b], gsem[b]).wait()

    def issue_write(h, b):
        pltpu.async_copy(rows[b],
                         out_hbm.at[h, pl.ds(base_b, BATCH_PER_W)], wsem[b])

    def wait_write(b):
        pltpu.make_async_copy(
            rows[b], out_hbm.at[0, pl.ds(0, BATCH_PER_W)], wsem[b]).wait()

    for b in range(NBUF):
        issue_gather(b, b)

    def group_body(g, _):
        for b in range(NBUF):
            wait_gather(b)
            issue_write(g * NBUF + b, b)
        for b in range(NBUF):
            wait_write(b)
            issue_gather((g + 1) * NBUF + b, b)
        return 0

    lax.fori_loop(0, NGROUPS - 1, group_body, 0)

    for b in range(NBUF):
        wait_gather(b)
        issue_write((NGROUPS - 1) * NBUF + b, b)
    for b in range(NBUF):
        wait_write(b)


def kernel(x, table):
    x_flat = x.astype(jnp.int32).reshape(B)
    out = _embed_gather(x_flat, table)
    return jnp.transpose(out, (1, 0, 2))


# speedup vs baseline: 9.9898x; 1.0144x over previous
import functools

import jax
import jax.numpy as jnp
from jax import lax
from jax.experimental import pallas as pl
from jax.experimental.pallas import tpu as pltpu
from jax.experimental.pallas import tpu_sc as plsc

BATCH = 4096
HIST = 50
DIM = 128

_info = plsc.get_sparse_core_info()
NC, NS, L = _info.num_cores, _info.num_subcores, _info.num_lanes
NW = NC * NS
B = BATCH * HIST
B_PER_W = B // NW
BATCH_PER_W = BATCH // NW
NBUF = 5
NGROUPS = HIST // NBUF

_mesh = plsc.VectorSubcoreMesh(core_axis_name="c", subcore_axis_name="s")


@functools.partial(
    pl.kernel,
    mesh=_mesh,
    out_type=jax.ShapeDtypeStruct((HIST, BATCH, DIM), jnp.float32),
    scratch_types=(
        [
            pltpu.VMEM((B_PER_W,), jnp.int32),
            pltpu.VMEM((B_PER_W,), jnp.int32),
        ]
        + [pltpu.VMEM((BATCH_PER_W, DIM), jnp.float32)] * NBUF
        + [pltpu.SemaphoreType.DMA] * (2 * NBUF)
    ),
    compiler_params=pltpu.CompilerParams(needs_layout_passes=False),
)
def _embed_gather(x_hbm, table_hbm, out_hbm, idx_flat, idx_str, *bufs):
    rows = bufs[:NBUF]
    gsem = bufs[NBUF:2 * NBUF]
    wsem = bufs[2 * NBUF:]
    wid = lax.axis_index("s") * NC + lax.axis_index("c")
    base_b = wid * BATCH_PER_W

    pltpu.sync_copy(x_hbm.at[pl.ds(wid * B_PER_W, B_PER_W)], idx_flat)

    lanes_h = lax.iota(jnp.int32, L) * HIST

    def rearrange(h):
        for l in range(BATCH_PER_W // L):
            src = lanes_h + (l * L * HIST + h)
            v = plsc.load_gather(idx_flat, [src])
            idx_str[pl.ds(h * BATCH_PER_W + l * L, L)] = v + 1

    def issue_gather(h, b):
        pltpu.async_copy(
            table_hbm.at[idx_str.at[pl.ds(h * BATCH_PER_W, BATCH_PER_W)]],
            rows[b], gsem[b])

    def wait_gather(b):
        pltpu.make_async_copy(
            table_hbm.at[pl.ds(0, BATCH_PER_W)], rows[b], gsem[b]).wait()

    def issue_write(h, b):
        pltpu.async_copy(rows[b],
                         out_hbm.at[h, pl.ds(base_b, BATCH_PER_W)], wsem[b])

    def wait_write(b):
        pltpu.make_async_copy(
            rows[b], out_hbm.at[0, pl.ds(0, BATCH_PER_W)], wsem[b]).wait()

    for b in range(NBUF):
        rearrange(b)
    for b in range(NBUF):
        issue_gather(b, b)

    def group_body(g, _):
        for b in range(NBUF):
            rearrange((g + 1) * NBUF + b)
        for b in range(NBUF):
            wait_gather(b)
            issue_write(g * NBUF + b, b)
        for b in range(NBUF):
            wait_write(b)
            issue_gather((g + 1) * NBUF + b, b)
        return 0

    lax.fori_loop(0, NGROUPS - 1, group_body, 0)

    for b in range(NBUF):
        wait_gather(b)
        issue_write((NGROUPS - 1) * NBUF + b, b)
    for b in range(NBUF):
        wait_write(b)


def kernel(x, table):
    x_flat = x.astype(jnp.int32).reshape(B)
    out = _embed_gather(x_flat, table)
    return jnp.transpose(out, (1, 0, 2))


# speedup vs baseline: 10.3442x; 1.0355x over previous
import functools

import jax
import jax.numpy as jnp
from jax import lax
from jax.experimental import pallas as pl
from jax.experimental.pallas import tpu as pltpu
from jax.experimental.pallas import tpu_sc as plsc

BATCH = 4096
HIST = 50
DIM = 128

_info = plsc.get_sparse_core_info()
NC, NS, L = _info.num_cores, _info.num_subcores, _info.num_lanes
NW = NC * NS
B = BATCH * HIST
BATCH_PER_W = BATCH // NW
NBUF = 5
NGROUPS = HIST // NBUF

_mesh = plsc.VectorSubcoreMesh(core_axis_name="c", subcore_axis_name="s")


@functools.partial(
    pl.kernel,
    mesh=_mesh,
    out_type=jax.ShapeDtypeStruct((HIST, BATCH, DIM), jnp.float32),
    scratch_types=(
        [pltpu.VMEM((HIST, BATCH_PER_W), jnp.int32)]
        + [pltpu.VMEM((BATCH_PER_W, DIM), jnp.float32)] * NBUF
        + [pltpu.SemaphoreType.DMA] * (2 * NBUF)
    ),
    compiler_params=pltpu.CompilerParams(needs_layout_passes=False),
)
def _embed_gather(xt_hbm, table_hbm, out_hbm, idx_v, *bufs):
    rows = bufs[:NBUF]
    gsem = bufs[NBUF:2 * NBUF]
    wsem = bufs[2 * NBUF:]
    wid = lax.axis_index("s") * NC + lax.axis_index("c")
    base_b = wid * BATCH_PER_W

    pltpu.sync_copy(xt_hbm.at[pl.ds(0, HIST), pl.ds(base_b, BATCH_PER_W)],
                    idx_v)

    def shift(h):
        for l in range(BATCH_PER_W // L):
            sl = pl.ds(l * L, L)
            idx_v[h, sl] = idx_v[h, sl] + 1

    def issue_gather(h, b):
        pltpu.async_copy(table_hbm.at[idx_v.at[h]], rows[b], gsem[b])

    def wait_gather(b):
        pltpu.make_async_copy(
            table_hbm.at[pl.ds(0, BATCH_PER_W)], rows[b], gsem[b]).wait()

    def issue_write(h, b):
        pltpu.async_copy(rows[b],
                         out_hbm.at[h, pl.ds(base_b, BATCH_PER_W)], wsem[b])

    def wait_write(b):
        pltpu.make_async_copy(
            rows[b], out_hbm.at[0, pl.ds(0, BATCH_PER_W)], wsem[b]).wait()

    for b in range(NBUF):
        shift(b)
    for b in range(NBUF):
        issue_gather(b, b)

    def group_body(g, _):
        for b in range(NBUF):
            shift((g + 1) * NBUF + b)
        for b in range(NBUF):
            wait_gather(b)
            issue_write(g * NBUF + b, b)
        for b in range(NBUF):
            wait_write(b)
            issue_gather((g + 1) * NBUF + b, b)
        return 0

    lax.fori_loop(0, NGROUPS - 1, group_body, 0)

    for b in range(NBUF):
        wait_gather(b)
        issue_write((NGROUPS - 1) * NBUF + b, b)
    for b in range(NBUF):
        wait_write(b)


def kernel(x, table):
    xt = jnp.transpose(x.astype(jnp.int32))
    out = _embed_gather(xt, table)
    return jnp.transpose(out, (1, 0, 2))
